# Initial kernel scaffold; baseline (speedup 1.0000x reference)
import jax
import jax.numpy as jnp
from jax.experimental import pallas as pl


def kernel(x, edge_index, edge_label_index, W1_rel, b1_rel, W1_root, W2_rel, b2_rel, W2_root):
    raise NotImplementedError("write your pallas kernel here")


# speedup vs baseline: 1.4499x; 1.4499x over previous
import functools
import math

import jax
import jax.numpy as jnp
from jax import lax
from jax.experimental import pallas as pl
from jax.experimental.pallas import tpu as pltpu
from jax.experimental.pallas import tpu_sc as plsc

NC = 2
NS = 16
NW = NC * NS
CHUNK = 128


def _make_agg(k_chunks, npad, d, with_counts):
  mesh = plsc.VectorSubcoreMesh(core_axis_name="c", subcore_axis_name="s",
                                num_cores=NC, num_subcores=NS)
  rows_per_tile = npad // NS
  crows = npad // CHUNK

  out_type = [jax.ShapeDtypeStruct((NC, npad, d), jnp.float32)]
  scratch = [
      pltpu.VMEM_SHARED((npad, d), jnp.float32),
      pltpu.VMEM((8, CHUNK), jnp.int32),
      pltpu.VMEM((8, CHUNK), jnp.int32),
      pltpu.VMEM((CHUNK, d), jnp.float32),
      pltpu.SemaphoreType.DMA,
      pltpu.SemaphoreType.DMA,
  ]
  if with_counts:
    out_type.append(jax.ShapeDtypeStruct((NW, crows, CHUNK), jnp.float32))
    scratch.append(pltpu.VMEM((crows, CHUNK), jnp.float32))

  def body(*refs):
    if with_counts:
      (h, src2, dst2, zrows, out, cnt_out,
       acc, srcbuf, dstbuf, rows, sem_g, sem_s, cntbuf) = refs
    else:
      (h, src2, dst2, zrows, out,
       acc, srcbuf, dstbuf, rows, sem_g, sem_s) = refs

    cid = lax.axis_index("c")
    sid = lax.axis_index("s")
    tid = cid * NS + sid
    zsl = pl.ds(sid * rows_per_tile, rows_per_tile)
    ones16 = jnp.ones((16,), jnp.float32)

    pltpu.sync_copy(zrows.at[zsl], acc.at[zsl])
    if with_counts:
      zero16 = jnp.zeros((16,), jnp.float32)

      @pl.loop(0, crows)
      def _(r):
        for q in range(CHUNK // 16):
          cntbuf[r, pl.ds(q * 16, 16)] = zero16
    base = tid * k_chunks
    plsc.subcore_barrier()

    @pl.loop(0, k_chunks // 8)
    def _(jj):
      off = pl.multiple_of(base + jj * 8, 8)
      pltpu.sync_copy(src2.at[pl.ds(off, 8)], srcbuf)
      pltpu.sync_copy(dst2.at[pl.ds(off, 8)], dstbuf)
      for t in range(8):
        pltpu.async_copy(h.at[srcbuf.at[t]], rows, sem_g).wait()
        pltpu.async_copy(rows, acc.at[dstbuf.at[t]], sem_s, add=True).wait()
        if with_counts:
          for i in range(CHUNK // 16):
            dv = dstbuf[t, pl.ds(i * 16, 16)]
            plsc.addupdate_scatter(
                cntbuf,
                [lax.shift_right_logical(dv, 7), jnp.bitwise_and(dv, 127)],
                ones16)

    plsc.subcore_barrier()
    pltpu.sync_copy(acc.at[zsl], out.at[cid, zsl])
    if with_counts:
      pltpu.sync_copy(cntbuf, cnt_out.at[tid])

  return pl.kernel(body, out_type=out_type, mesh=mesh, scratch_types=scratch,
                   compiler_params=pltpu.CompilerParams(
                       needs_layout_passes=False),
                   name="sc_seg_sum" + ("_cnt" if with_counts else ""))


def _make_dense(n, npad, d, relu):
  crows = npad // CHUNK

  def body(p_ref, c_ref, h_ref, wrel_ref, b_ref, wroot_ref, o_ref):
    s = p_ref[0, :, :] + p_ref[1, :, :]
    cnt = jnp.maximum(jnp.sum(c_ref[:, :, :], axis=0), 1.0)
    mean = (s.reshape(crows, CHUNK, d)
            / cnt.reshape(crows, CHUNK, 1)).reshape(npad, d)
    z = (jnp.dot(mean[0:n, :], wrel_ref[:, :],
                 preferred_element_type=jnp.float32)
         + b_ref[:, :]
         + jnp.dot(h_ref[:, :], wroot_ref[:, :],
                   preferred_element_type=jnp.float32))
    o_ref[:, :] = jnp.maximum(z, 0.0) if relu else z

  return pl.pallas_call(
      body, out_shape=jax.ShapeDtypeStruct((n, d), jnp.float32))


def _make_decode(k_chunks, d, scale):
  mesh = plsc.VectorSubcoreMesh(core_axis_name="c", subcore_axis_name="s",
                                num_cores=NC, num_subcores=NS)
  ngroups = CHUNK // 16

  @functools.partial(
      pl.kernel,
      out_type=jax.ShapeDtypeStruct((NW * k_chunks, CHUNK), jnp.float32),
      mesh=mesh,
      scratch_types=[
          pltpu.VMEM((k_chunks, CHUNK), jnp.int32),
          pltpu.VMEM((k_chunks, CHUNK), jnp.int32),
          pltpu.VMEM((CHUNK, d), jnp.float32),
          pltpu.VMEM((CHUNK, d), jnp.float32),
          pltpu.VMEM((k_chunks, CHUNK), jnp.float32),
          pltpu.SemaphoreType.DMA,
          pltpu.SemaphoreType.DMA,
      ],
      compiler_params=pltpu.CompilerParams(needs_layout_passes=False),
      name="sc_decode",
  )
  def decode(z, a2, b2, out, abuf, bbuf, s_rows, d_rows, obuf, sem_a, sem_b):
    cid = lax.axis_index("c")
    sid = lax.axis_index("s")
    tid = cid * NS + sid
    base = tid * k_chunks
    pltpu.sync_copy(a2.at[pl.ds(base, k_chunks)], abuf)
    pltpu.sync_copy(b2.at[pl.ds(base, k_chunks)], bbuf)
    lane = jnp.arange(16, dtype=jnp.int32)
    evecs = [lane + g * 16 for g in range(ngroups)]
    zero16 = jnp.zeros((16,), jnp.float32)

    @pl.loop(0, k_chunks)
    def _(j):
      ca = pltpu.async_copy(z.at[abuf.at[j]], s_rows, sem_a)
      cb = pltpu.async_copy(z.at[bbuf.at[j]], d_rows, sem_b)
      ca.wait()
      cb.wait()

      @pl.loop(0, d, init_carry=(zero16,) * ngroups)
      def res8(c, carry):
        col = jnp.full((16,), c, jnp.int32)
        out_c = []
        for g in range(ngroups):
          sv = plsc.load_gather(s_rows, [evecs[g], col])
          dv = plsc.load_gather(d_rows, [evecs[g], col])
          out_c.append(carry[g] + sv * dv)
        return tuple(out_c)

      for g in range(ngroups):
        obuf[j, pl.ds(g * 16, 16)] = res8[g] * scale

    pltpu.sync_copy(obuf, out.at[pl.ds(tid * k_chunks, k_chunks)])

  return decode


def kernel(x, edge_index, edge_label_index, W1_rel, b1_rel, W1_root,
           W2_rel, b2_rel, W2_root):
  n, d = x.shape
  e = edge_index.shape[1]
  el = edge_label_index.shape[1]
  f32 = jnp.float32

  npad = ((n + 1 + 127) // 128) * 128
  ke = ((-(-e // (NW * CHUNK)) + 7) // 8) * 8
  epad = ke * NW * CHUNK
  kl = ((-(-el // (NW * CHUNK)) + 7) // 8) * 8
  elpad = kl * NW * CHUNK

  pad_e = jnp.full((epad - e,), n, jnp.int32)
  src2 = jnp.concatenate([edge_index[0], pad_e]).reshape(epad // CHUNK, CHUNK)
  dst2 = jnp.concatenate([edge_index[1], pad_e]).reshape(epad // CHUNK, CHUNK)
  pad_l = jnp.zeros((elpad - el,), jnp.int32)
  a2 = jnp.concatenate([edge_label_index[0], pad_l]).reshape(
      elpad // CHUNK, CHUNK)
  b2 = jnp.concatenate([edge_label_index[1], pad_l]).reshape(
      elpad // CHUNK, CHUNK)

  zrows = jnp.zeros((npad, d), f32)
  zpad_n = jnp.zeros((npad - n, d), f32)

  x_pad = jnp.concatenate([x, zpad_n], axis=0)
  p1, cnt = _make_agg(ke, npad, d, True)(x_pad, src2, dst2, zrows)
  z1 = _make_dense(n, npad, d, True)(
      p1, cnt, x, W1_rel.T, b1_rel.reshape(1, d), W1_root.T)

  z1_pad = jnp.concatenate([z1, zpad_n], axis=0)
  (p2,) = _make_agg(ke, npad, d, False)(z1_pad, src2, dst2, zrows)
  z2 = _make_dense(n, npad, d, False)(
      p2, cnt, z1, W2_rel.T, b2_rel.reshape(1, d), W2_root.T)

  raw = _make_decode(kl, d, 1.0 / math.sqrt(d))(z2, a2, b2)
  return raw.reshape(-1)[:el]
